# Initial kernel scaffold; baseline (speedup 1.0000x reference)
import jax
import jax.numpy as jnp
from jax.experimental import pallas as pl


def kernel(x, edge_index, batch, g1w1, g1b1, g1w2, g1b2, g2w1, g2b1, g2w2, g2b2, g3w1, g3b1, g3w2, g3b2, g4w1, g4b1, g4w2, g4b2, g5w1, g5b1, g5w2, g5b2, lin_w, lin_b, ln_g, ln_b):
    raise NotImplementedError("write your pallas kernel here")


# speedup vs baseline: 4.5111x; 4.5111x over previous
import functools

import jax
import jax.numpy as jnp
from jax import lax
from jax.experimental import pallas as pl
from jax.experimental.pallas import tpu as pltpu
from jax.experimental.pallas import tpu_sc as plsc

N_NODES = 10000
N_EDGES = 320000
D = 128
N_GRAPHS = 64
NEG_SLOPE = (1.0 / 8.0 + 1.0 / 3.0) / 2.0

NC = 2
NS = 16
NW = NC * NS
EDGES_PER_W = N_EDGES // NW
CHUNK = 80
N_CHUNK = EDGES_PER_W // CHUNK
ACC_ROWS = 10240
TROWS = ACC_ROWS // NS
ZROWS = 128
TAIL_ROWS = N_NODES - (NS - 1) * TROWS


def _agg_body(x_hbm, src_hbm, dst_hbm, out_hbm,
              idx_s, idx_d, rows, zbuf, acc, sem):
    cid = lax.axis_index("c")
    sid = lax.axis_index("s")
    wid = sid * NC + cid

    zvec = jnp.zeros((16,), jnp.float32)

    def zrow(i, _):
        for k in range(D // 16):
            zbuf[i, pl.ds(k * 16, 16)] = zvec
        return 0

    lax.fori_loop(0, ZROWS, zrow, 0)
    trow0 = pl.multiple_of(sid * TROWS, 8)
    for r in range(TROWS // ZROWS):
        pltpu.sync_copy(zbuf, acc.at[pl.ds(trow0 + r * ZROWS, ZROWS)])
    plsc.subcore_barrier()

    base0 = pl.multiple_of(wid * EDGES_PER_W, 8)

    def edge_step(i, _):
        base = pl.multiple_of(base0 + i * CHUNK, 8)
        pltpu.sync_copy(src_hbm.at[pl.ds(base, CHUNK)], idx_s)
        pltpu.sync_copy(dst_hbm.at[pl.ds(base, CHUNK)], idx_d)
        pltpu.async_copy(x_hbm.at[idx_s], rows, sem).wait()
        pltpu.sync_copy(rows, acc.at[idx_d], add=True)
        return 0

    lax.fori_loop(0, N_CHUNK, edge_step, 0)
    plsc.subcore_barrier()

    @pl.when(sid < NS - 1)
    def _drain_main():
        pltpu.sync_copy(acc.at[pl.ds(trow0, TROWS)],
                        out_hbm.at[cid, pl.ds(trow0, TROWS)])

    @pl.when(sid == NS - 1)
    def _drain_tail():
        pltpu.sync_copy(acc.at[pl.ds(trow0, TAIL_ROWS)],
                        out_hbm.at[cid, pl.ds(trow0, TAIL_ROWS)])


@functools.cache
def _make_agg():
    return pl.kernel(
        _agg_body,
        out_type=jax.ShapeDtypeStruct((NC, N_NODES, D), jnp.float32),
        mesh=plsc.VectorSubcoreMesh(core_axis_name="c", subcore_axis_name="s",
                                    num_cores=NC, num_subcores=NS),
        scratch_types=[
            pltpu.VMEM((CHUNK,), jnp.int32),
            pltpu.VMEM((CHUNK,), jnp.int32),
            pltpu.VMEM((CHUNK, D), jnp.float32),
            pltpu.VMEM((ZROWS, D), jnp.float32),
            pltpu.VMEM_SHARED((ACC_ROWS, D), jnp.float32),
            pltpu.SemaphoreType.DMA,
        ],
    )


def _agg(x, src, dst):
    return _make_agg()(x, src, dst)


ROW_BLK = 1000


def _mlp_body(x_ref, parts_ref, w1_ref, b1_ref, w2_ref, b2_ref, o_ref):
    h = x_ref[...] + parts_ref[0] + parts_ref[1]
    h = jnp.dot(h, w1_ref[...], preferred_element_type=jnp.float32)
    h = jnp.maximum(h + b1_ref[...], 0.0)
    h = jnp.dot(h, w2_ref[...], preferred_element_type=jnp.float32)
    h = h + b2_ref[...]
    o_ref[...] = jnp.where(h >= 0, h, h * NEG_SLOPE)


def _mlp(x, parts, w1, b1, w2, b2):
    return pl.pallas_call(
        _mlp_body,
        grid=(N_NODES // ROW_BLK,),
        in_specs=[
            pl.BlockSpec((ROW_BLK, D), lambda i: (i, 0)),
            pl.BlockSpec((NC, ROW_BLK, D), lambda i: (0, i, 0)),
            pl.BlockSpec((D, D), lambda i: (0, 0)),
            pl.BlockSpec((1, D), lambda i: (0, 0)),
            pl.BlockSpec((D, D), lambda i: (0, 0)),
            pl.BlockSpec((1, D), lambda i: (0, 0)),
        ],
        out_specs=pl.BlockSpec((ROW_BLK, D), lambda i: (i, 0)),
        out_shape=jax.ShapeDtypeStruct((N_NODES, D), jnp.float32),
    )(x, parts, w1, b1, w2, b2)


def _pool_body(x_ref, batch_ref, lin_w_ref, lin_b_ref, ln_g_ref, ln_b_ref,
               o_ref):
    xv = x_ref[...]
    b = batch_ref[...]
    gids = lax.broadcasted_iota(jnp.int32, (1, N_GRAPHS), 1)
    oh = (b == gids).astype(jnp.float32)
    pooled = lax.dot_general(oh, xv, (((0,), (0,)), ((), ())))
    ones = jnp.ones((N_NODES, 1), jnp.float32)
    counts = lax.dot_general(oh, ones, (((0,), (0,)), ((), ())))
    y = jnp.dot(pooled, lin_w_ref[...], preferred_element_type=jnp.float32)
    y = y + counts * lin_b_ref[...]
    mu = jnp.mean(y, axis=1, keepdims=True)
    var = jnp.mean((y - mu) ** 2, axis=1, keepdims=True)
    o_ref[...] = (y - mu) * lax.rsqrt(var + 1e-5) * ln_g_ref[...] + ln_b_ref[...]


def _pool(x, batch2d, lin_w, lin_b, ln_g, ln_b):
    return pl.pallas_call(
        _pool_body,
        out_shape=jax.ShapeDtypeStruct((N_GRAPHS, D), jnp.float32),
    )(x, batch2d, lin_w, lin_b, ln_g, ln_b)


def kernel(x, edge_index, batch, g1w1, g1b1, g1w2, g1b2, g2w1, g2b1, g2w2,
           g2b2, g3w1, g3b1, g3w2, g3b2, g4w1, g4b1, g4w2, g4b2, g5w1, g5b1,
           g5w2, g5b2, lin_w, lin_b, ln_g, ln_b):
    src = edge_index[0].astype(jnp.int32)
    dst = edge_index[1].astype(jnp.int32)
    layers = [
        (g1w1, g1b1, g1w2, g1b2),
        (g2w1, g2b1, g2w2, g2b2),
        (g3w1, g3b1, g3w2, g3b2),
        (g4w1, g4b1, g4w2, g4b2),
        (g5w1, g5b1, g5w2, g5b2),
    ]
    for w1, b1, w2, b2 in layers:
        parts = _agg(x, src, dst)
        x = _mlp(x, parts, w1, b1.reshape(1, D), w2, b2.reshape(1, D))
    return _pool(x, batch.astype(jnp.int32).reshape(N_NODES, 1), lin_w,
                 lin_b.reshape(1, D), ln_g.reshape(1, D), ln_b.reshape(1, D))
